# baseline (speedup 1.0000x reference)
import functools

import jax
import jax.numpy as jnp
from jax import lax
from jax.experimental import pallas as pl
from jax.experimental.pallas import tpu as pltpu
from jax.experimental.pallas import tpu_sc as plsc

_NC = 2
_NS = 16
_NW = _NC * _NS
_CH = 128


def _rup(a, b):
    return (a + b - 1) // b * b


@functools.lru_cache(maxsize=None)
def _make_deg_kernel(NP, NCH):
    RT = NP // _NS
    mesh = plsc.VectorSubcoreMesh(core_axis_name="c", subcore_axis_name="s")

    @functools.partial(
        pl.kernel,
        out_type=jax.ShapeDtypeStruct((_NC, NP, 16), jnp.float32),
        mesh=mesh,
        scratch_types=[
            pltpu.VMEM((_CH,), jnp.int32),
            pltpu.VMEM((_CH,), jnp.int32),
            pltpu.VMEM((_CH, 16), jnp.float32),
            pltpu.VMEM_SHARED((NP, 16), jnp.float32),
            pltpu.SemaphoreType.DMA,
            pltpu.SemaphoreType.DMA,
            pltpu.SemaphoreType.DMA,
            pltpu.SemaphoreType.DMA,
        ],
    )
    def kdeg(dstr_hbm, ones_hbm, zrow_hbm, out_hbm,
             dst0, dst1, ones_v, acc_sh, sem_d0, sem_d1, sem_s0, sem_s1):
        c = lax.axis_index("c")
        s = lax.axis_index("s")
        wid = s * _NC + c
        dsts = (dst0, dst1)
        sem_d = (sem_d0, sem_d1)
        sem_s = (sem_s0, sem_s1)
        pltpu.sync_copy(ones_hbm, ones_v)
        pltpu.sync_copy(zrow_hbm, acc_sh.at[pl.ds(s * RT, RT)])
        plsc.subcore_barrier()

        pltpu.async_copy(dstr_hbm.at[wid, 0], dsts[0], sem_d[0])

        def pair_body(p, carry):
            for S in (0, 1):
                j = 2 * p + S
                SN = 1 - S

                @pl.when(j + 1 < NCH)
                def _():
                    @pl.when(j >= 1)
                    def _():
                        pltpu.make_async_copy(
                            ones_v, acc_sh.at[dsts[SN]], sem_s[SN]).wait()
                    pltpu.async_copy(dstr_hbm.at[wid, j + 1], dsts[SN],
                                     sem_d[SN])

                pltpu.make_async_copy(dstr_hbm.at[wid, j], dsts[S],
                                      sem_d[S]).wait()
                pltpu.async_copy(ones_v, acc_sh.at[dsts[S]], sem_s[S],
                                 add=True)
            return carry

        lax.fori_loop(0, NCH // 2, pair_body, 0)
        for S in (0, 1):
            pltpu.make_async_copy(ones_v, acc_sh.at[dsts[S]], sem_s[S]).wait()
        plsc.subcore_barrier()
        pltpu.sync_copy(acc_sh.at[pl.ds(s * RT, RT)],
                        out_hbm.at[c, pl.ds(s * RT, RT)])

    return kdeg


@functools.lru_cache(maxsize=None)
def _make_edge_scatter_kernel(NP, Hd, NCH):
    RT = NP // _NS
    mesh = plsc.VectorSubcoreMesh(core_axis_name="c", subcore_axis_name="s")

    @functools.partial(
        pl.kernel,
        out_type=jax.ShapeDtypeStruct((_NC, NP, Hd), jnp.float32),
        mesh=mesh,
        scratch_types=[
            pltpu.VMEM((NCH, _CH), jnp.int32),
            pltpu.VMEM((_CH,), jnp.int32),
            pltpu.VMEM((_CH,), jnp.int32),
            pltpu.VMEM((_CH, Hd), jnp.float32),
            pltpu.VMEM((_CH, Hd), jnp.float32),
            pltpu.VMEM_SHARED((NP, Hd), jnp.float32),
            pltpu.SemaphoreType.DMA,
            pltpu.SemaphoreType.DMA,
            pltpu.SemaphoreType.DMA,
            pltpu.SemaphoreType.DMA,
            pltpu.SemaphoreType.DMA,
            pltpu.SemaphoreType.DMA,
        ],
    )
    def kscat(hs_hbm, srcr_hbm, dstr_hbm, zrow_hbm, out_hbm,
              src_all, dst0, dst1, rows0, rows1, acc_sh,
              sem_d0, sem_d1, sem_g0, sem_g1, sem_s0, sem_s1):
        c = lax.axis_index("c")
        s = lax.axis_index("s")
        wid = s * _NC + c
        dsts = (dst0, dst1)
        rows = (rows0, rows1)
        sem_d = (sem_d0, sem_d1)
        sem_g = (sem_g0, sem_g1)
        sem_s = (sem_s0, sem_s1)
        pltpu.sync_copy(zrow_hbm, acc_sh.at[pl.ds(s * RT, RT)])
        pltpu.sync_copy(srcr_hbm.at[wid], src_all)
        plsc.subcore_barrier()

        def prefetch(S, j):
            pltpu.async_copy(dstr_hbm.at[wid, j], dsts[S], sem_d[S])
            pltpu.async_copy(hs_hbm.at[src_all.at[j]], rows[S], sem_g[S])

        prefetch(0, 0)

        def pair_body(p, carry):
            for S in (0, 1):
                j = 2 * p + S
                SN = 1 - S

                @pl.when(j + 1 < NCH)
                def _():
                    @pl.when(j >= 1)
                    def _():
                        pltpu.make_async_copy(
                            rows[SN], acc_sh.at[dsts[SN]], sem_s[SN]).wait()
                    prefetch(SN, j + 1)

                pltpu.make_async_copy(dstr_hbm.at[wid, j], dsts[S],
                                      sem_d[S]).wait()
                pltpu.make_async_copy(hs_hbm.at[src_all.at[j]], rows[S],
                                      sem_g[S]).wait()
                pltpu.async_copy(rows[S], acc_sh.at[dsts[S]], sem_s[S],
                                 add=True)
            return carry

        lax.fori_loop(0, NCH // 2, pair_body, 0)
        for S in (0, 1):
            pltpu.make_async_copy(rows[S], acc_sh.at[dsts[S]], sem_s[S]).wait()
        plsc.subcore_barrier()
        pltpu.sync_copy(acc_sh.at[pl.ds(s * RT, RT)],
                        out_hbm.at[c, pl.ds(s * RT, RT)])

    return kscat


def _prep_body(x_ref, w1_ref, degp_ref, hs_ref, dinv_ref):
    x = x_ref[...]
    rowsum = jnp.sum(x, axis=1, keepdims=True)
    rinv = jnp.where(rowsum != 0.0, 1.0 / rowsum, 0.0)
    xn = x * rinv
    deg = degp_ref[0, :, 0:1] + degp_ref[1, :, 0:1] + 1.0
    dinv = lax.rsqrt(deg)
    h = jnp.dot(xn, w1_ref[...], preferred_element_type=jnp.float32)
    hs_ref[...] = h * dinv
    dinv_ref[...] = dinv


def _mid_body(accp_ref, hs_ref, dinv_ref, b_ref, w2_ref, hs2_ref):
    dinv = dinv_ref[...]
    out1 = (accp_ref[0] + accp_ref[1] + hs_ref[...]) * dinv + b_ref[...]
    h1 = jnp.maximum(out1, 0.0)
    h2 = jnp.dot(h1, w2_ref[...], preferred_element_type=jnp.float32)
    hs2_ref[...] = h2 * dinv


def _fin_body(accp_ref, hs_ref, dinv_ref, b_ref, out_ref):
    out_ref[...] = ((accp_ref[0] + accp_ref[1] + hs_ref[...]) * dinv_ref[...]
                    + b_ref[...])


def kernel(x, edge_index, W1, b1, W2, b2, group):
    N, D = x.shape
    H = W1.shape[1]
    E = edge_index.shape[1]

    NP = _rup(N + 1, _NS * 8)
    EW = _rup(-(-E // _NW), 4 * _CH)
    NCH = EW // _CH

    pad_e = EW * _NW - E
    src = jnp.concatenate(
        [edge_index[0], jnp.full((pad_e,), N, jnp.int32)]).reshape(_NW, NCH, _CH)
    dst = jnp.concatenate(
        [edge_index[1], jnp.full((pad_e,), N, jnp.int32)]).reshape(_NW, NCH, _CH)

    xpad = jnp.zeros((NP, D), jnp.float32).at[:N].set(x)
    RT = NP // _NS
    zrow16 = jnp.zeros((RT, 16), jnp.float32)
    zrowH = jnp.zeros((RT, H), jnp.float32)
    ones16 = jnp.ones((_CH, 16), jnp.float32)
    b1r = b1.reshape(1, H)
    b2r = b2.reshape(1, H)

    degp = _make_deg_kernel(NP, NCH)(dst, ones16, zrow16)

    hs1, dinv = pl.pallas_call(
        _prep_body,
        out_shape=(
            jax.ShapeDtypeStruct((NP, H), jnp.float32),
            jax.ShapeDtypeStruct((NP, 1), jnp.float32),
        ),
    )(xpad, W1, degp)

    edge_scatter = _make_edge_scatter_kernel(NP, H, NCH)
    acc1 = edge_scatter(hs1, src, dst, zrowH)

    hs2 = pl.pallas_call(
        _mid_body,
        out_shape=jax.ShapeDtypeStruct((NP, H), jnp.float32),
    )(acc1, hs1, dinv, b1r, W2)

    acc2 = edge_scatter(hs2, src, dst, zrowH)

    out = pl.pallas_call(
        _fin_body,
        out_shape=jax.ShapeDtypeStruct((NP, H), jnp.float32),
    )(acc2, hs2, dinv, b2r)

    return out[:N]


# speedup vs baseline: 3.1245x; 3.1245x over previous
import functools

import jax
import jax.numpy as jnp
from jax import lax
from jax.experimental import pallas as pl
from jax.experimental.pallas import tpu as pltpu
from jax.experimental.pallas import tpu_sc as plsc

_NC = 2
_NS = 16
_NW = _NC * _NS
_CH = 128


def _rup(a, b):
    return (a + b - 1) // b * b


@functools.lru_cache(maxsize=None)
def _make_deg_kernel(NP, NCH):
    RT = NP // _NS
    mesh = plsc.VectorSubcoreMesh(core_axis_name="c", subcore_axis_name="s")

    @functools.partial(
        pl.kernel,
        out_type=jax.ShapeDtypeStruct((_NC, NP, 16), jnp.float32),
        mesh=mesh,
        scratch_types=[
            pltpu.VMEM((_CH,), jnp.int32),
            pltpu.VMEM((_CH,), jnp.int32),
            pltpu.VMEM((_CH, 16), jnp.float32),
            pltpu.VMEM_SHARED((NP, 16), jnp.float32),
            pltpu.SemaphoreType.DMA,
            pltpu.SemaphoreType.DMA,
            pltpu.SemaphoreType.DMA,
            pltpu.SemaphoreType.DMA,
        ],
    )
    def kdeg(dstr_hbm, ones_hbm, zrow_hbm, out_hbm,
             dst0, dst1, ones_v, acc_sh, sem_d0, sem_d1, sem_s0, sem_s1):
        c = lax.axis_index("c")
        s = lax.axis_index("s")
        wid = s * _NC + c
        dsts = (dst0, dst1)
        sem_d = (sem_d0, sem_d1)
        sem_s = (sem_s0, sem_s1)
        pltpu.sync_copy(ones_hbm, ones_v)
        pltpu.sync_copy(zrow_hbm, acc_sh.at[pl.ds(s * RT, RT)])
        plsc.subcore_barrier()

        pltpu.async_copy(dstr_hbm.at[wid, 0], dsts[0], sem_d[0])

        def pair_body(p, carry):
            for S in (0, 1):
                j = 2 * p + S
                SN = 1 - S

                @pl.when(j + 1 < NCH)
                def _():
                    @pl.when(j >= 1)
                    def _():
                        pltpu.make_async_copy(
                            ones_v, acc_sh.at[dsts[SN]], sem_s[SN]).wait()
                    pltpu.async_copy(dstr_hbm.at[wid, j + 1], dsts[SN],
                                     sem_d[SN])

                pltpu.make_async_copy(dstr_hbm.at[wid, j], dsts[S],
                                      sem_d[S]).wait()
                pltpu.async_copy(ones_v, acc_sh.at[dsts[S]], sem_s[S],
                                 add=True)
            return carry

        lax.fori_loop(0, NCH // 2, pair_body, 0)
        for S in (0, 1):
            pltpu.make_async_copy(ones_v, acc_sh.at[dsts[S]], sem_s[S]).wait()
        plsc.subcore_barrier()
        pltpu.sync_copy(acc_sh.at[pl.ds(s * RT, RT)],
                        out_hbm.at[c, pl.ds(s * RT, RT)])

    return kdeg


@functools.lru_cache(maxsize=None)
def _make_edge_scatter_kernel(NP, Hd, NCH):
    RT = NP // _NS
    mesh = plsc.VectorSubcoreMesh(core_axis_name="c", subcore_axis_name="s")

    @functools.partial(
        pl.kernel,
        out_type=jax.ShapeDtypeStruct((_NC, NP, Hd), jnp.float32),
        mesh=mesh,
        scratch_types=[
            pltpu.VMEM((NCH, _CH), jnp.int32),
            pltpu.VMEM((_CH,), jnp.int32),
            pltpu.VMEM((_CH,), jnp.int32),
            pltpu.VMEM((_CH, Hd), jnp.float32),
            pltpu.VMEM((_CH, Hd), jnp.float32),
            pltpu.VMEM_SHARED((NP, Hd), jnp.float32),
            pltpu.SemaphoreType.DMA,
            pltpu.SemaphoreType.DMA,
            pltpu.SemaphoreType.DMA,
            pltpu.SemaphoreType.DMA,
            pltpu.SemaphoreType.DMA,
            pltpu.SemaphoreType.DMA,
        ],
    )
    def kscat(hs_hbm, srcr_hbm, dstr_hbm, zrow_hbm, out_hbm,
              src_all, dst0, dst1, rows0, rows1, acc_sh,
              sem_d0, sem_d1, sem_g0, sem_g1, sem_s0, sem_s1):
        c = lax.axis_index("c")
        s = lax.axis_index("s")
        wid = s * _NC + c
        dsts = (dst0, dst1)
        rows = (rows0, rows1)
        sem_d = (sem_d0, sem_d1)
        sem_g = (sem_g0, sem_g1)
        sem_s = (sem_s0, sem_s1)
        pltpu.sync_copy(zrow_hbm, acc_sh.at[pl.ds(s * RT, RT)])
        pltpu.sync_copy(srcr_hbm.at[wid], src_all)
        plsc.subcore_barrier()

        def prefetch(S, j):
            pltpu.async_copy(dstr_hbm.at[wid, j], dsts[S], sem_d[S])
            pltpu.async_copy(hs_hbm.at[src_all.at[j]], rows[S], sem_g[S])

        prefetch(0, 0)

        def pair_body(p, carry):
            for S in (0, 1):
                j = 2 * p + S
                SN = 1 - S

                @pl.when(j + 1 < NCH)
                def _():
                    @pl.when(j >= 1)
                    def _():
                        pltpu.make_async_copy(
                            rows[SN], acc_sh.at[dsts[SN]], sem_s[SN]).wait()
                    prefetch(SN, j + 1)

                pltpu.make_async_copy(dstr_hbm.at[wid, j], dsts[S],
                                      sem_d[S]).wait()
                pltpu.make_async_copy(hs_hbm.at[src_all.at[j]], rows[S],
                                      sem_g[S]).wait()
                pltpu.async_copy(rows[S], acc_sh.at[dsts[S]], sem_s[S],
                                 add=True)
            return carry

        lax.fori_loop(0, NCH // 2, pair_body, 0)
        for S in (0, 1):
            pltpu.make_async_copy(rows[S], acc_sh.at[dsts[S]], sem_s[S]).wait()
        plsc.subcore_barrier()
        pltpu.sync_copy(acc_sh.at[pl.ds(s * RT, RT)],
                        out_hbm.at[c, pl.ds(s * RT, RT)])

    return kscat


def _prep_body(x_ref, w1_ref, degp_ref, hs_ref, dinv_ref):
    x = x_ref[...]
    rowsum = jnp.sum(x, axis=1, keepdims=True)
    rinv = jnp.where(rowsum != 0.0, 1.0 / rowsum, 0.0)
    xn = x * rinv
    deg = degp_ref[0, :, 0:1] + degp_ref[1, :, 0:1] + 1.0
    dinv = lax.rsqrt(deg)
    h = jnp.dot(xn, w1_ref[...], preferred_element_type=jnp.float32)
    hs_ref[...] = h * dinv
    dinv_ref[...] = dinv


def _mid_body(accp_ref, hs_ref, dinv_ref, b_ref, w2_ref, hs2_ref):
    dinv = dinv_ref[...]
    out1 = (accp_ref[0] + accp_ref[1] + hs_ref[...]) * dinv + b_ref[...]
    h1 = jnp.maximum(out1, 0.0)
    h2 = jnp.dot(h1, w2_ref[...], preferred_element_type=jnp.float32)
    hs2_ref[...] = h2 * dinv


def _fin_body(accp_ref, hs_ref, dinv_ref, b_ref, out_ref):
    out_ref[...] = ((accp_ref[0] + accp_ref[1] + hs_ref[...]) * dinv_ref[...]
                    + b_ref[...])


def kernel(x, edge_index, W1, b1, W2, b2, group):
    N, D = x.shape
    H = W1.shape[1]
    E = edge_index.shape[1]

    NP = _rup(N + 1, _NS * 8)
    EW = _rup(-(-E // _NW), 4 * _CH)
    NCH = EW // _CH

    pad_e = EW * _NW - E
    pad_idx = N + jnp.arange(pad_e, dtype=jnp.int32) % (NP - N)
    src = jnp.concatenate([edge_index[0], pad_idx]).reshape(_NW, NCH, _CH)
    dst = jnp.concatenate([edge_index[1], pad_idx]).reshape(_NW, NCH, _CH)

    xpad = jnp.zeros((NP, D), jnp.float32).at[:N].set(x)
    RT = NP // _NS
    zrow16 = jnp.zeros((RT, 16), jnp.float32)
    zrowH = jnp.zeros((RT, H), jnp.float32)
    ones16 = jnp.ones((_CH, 16), jnp.float32)
    b1r = b1.reshape(1, H)
    b2r = b2.reshape(1, H)

    degp = _make_deg_kernel(NP, NCH)(dst, ones16, zrow16)

    hs1, dinv = pl.pallas_call(
        _prep_body,
        out_shape=(
            jax.ShapeDtypeStruct((NP, H), jnp.float32),
            jax.ShapeDtypeStruct((NP, 1), jnp.float32),
        ),
    )(xpad, W1, degp)

    edge_scatter = _make_edge_scatter_kernel(NP, H, NCH)
    acc1 = edge_scatter(hs1, src, dst, zrowH)

    hs2 = pl.pallas_call(
        _mid_body,
        out_shape=jax.ShapeDtypeStruct((NP, H), jnp.float32),
    )(acc1, hs1, dinv, b1r, W2)

    acc2 = edge_scatter(hs2, src, dst, zrowH)

    out = pl.pallas_call(
        _fin_body,
        out_shape=jax.ShapeDtypeStruct((NP, H), jnp.float32),
    )(acc2, hs2, dinv, b2r)

    return out[:N]


# speedup vs baseline: 3.5457x; 1.1348x over previous
import functools

import jax
import jax.numpy as jnp
from jax import lax
from jax.experimental import pallas as pl
from jax.experimental.pallas import tpu as pltpu
from jax.experimental.pallas import tpu_sc as plsc

_NC = 2
_NS = 16
_NW = _NC * _NS
_CH = 128


def _rup(a, b):
    return (a + b - 1) // b * b


@functools.lru_cache(maxsize=None)
def _make_deg_kernel(NP, NCH):
    mesh = plsc.VectorSubcoreMesh(core_axis_name="c", subcore_axis_name="s")

    @functools.partial(
        pl.kernel,
        out_type=jax.ShapeDtypeStruct((_NW, NP), jnp.float32),
        mesh=mesh,
        compiler_params=pltpu.CompilerParams(needs_layout_passes=False),
        scratch_types=[
            pltpu.VMEM((NCH, _CH), jnp.int32),
            pltpu.VMEM((NP,), jnp.float32),
        ],
    )
    def kdeg(dstr_hbm, zfull_hbm, out_hbm, dst_all, deg_v):
        c = lax.axis_index("c")
        s = lax.axis_index("s")
        wid = s * _NC + c
        ones = jnp.ones((16,), jnp.float32)
        pltpu.sync_copy(zfull_hbm, deg_v)
        pltpu.sync_copy(dstr_hbm.at[wid], dst_all)

        def body(i, carry):
            for t in range(_CH // 16):
                idx = dst_all[i, pl.ds(t * 16, 16)]
                plsc.addupdate_scatter(deg_v, [idx], ones)
            return carry

        lax.fori_loop(0, NCH, body, 0)
        pltpu.sync_copy(deg_v, out_hbm.at[wid])

    return kdeg


@functools.lru_cache(maxsize=None)
def _make_edge_scatter_kernel(NP, Hd, NCH):
    RT = NP // _NS
    mesh = plsc.VectorSubcoreMesh(core_axis_name="c", subcore_axis_name="s")

    @functools.partial(
        pl.kernel,
        out_type=jax.ShapeDtypeStruct((_NC, NP, Hd), jnp.float32),
        mesh=mesh,
        scratch_types=[
            pltpu.VMEM((NCH, _CH), jnp.int32),
            pltpu.VMEM((_CH,), jnp.int32),
            pltpu.VMEM((_CH,), jnp.int32),
            pltpu.VMEM((_CH, Hd), jnp.float32),
            pltpu.VMEM((_CH, Hd), jnp.float32),
            pltpu.VMEM_SHARED((NP, Hd), jnp.float32),
            pltpu.SemaphoreType.DMA,
            pltpu.SemaphoreType.DMA,
            pltpu.SemaphoreType.DMA,
            pltpu.SemaphoreType.DMA,
            pltpu.SemaphoreType.DMA,
            pltpu.SemaphoreType.DMA,
        ],
    )
    def kscat(hs_hbm, srcr_hbm, dstr_hbm, zrow_hbm, out_hbm,
              src_all, dst0, dst1, rows0, rows1, acc_sh,
              sem_d0, sem_d1, sem_g0, sem_g1, sem_s0, sem_s1):
        c = lax.axis_index("c")
        s = lax.axis_index("s")
        wid = s * _NC + c
        dsts = (dst0, dst1)
        rows = (rows0, rows1)
        sem_d = (sem_d0, sem_d1)
        sem_g = (sem_g0, sem_g1)
        sem_s = (sem_s0, sem_s1)
        pltpu.sync_copy(zrow_hbm, acc_sh.at[pl.ds(s * RT, RT)])
        pltpu.sync_copy(srcr_hbm.at[wid], src_all)
        plsc.subcore_barrier()

        def prefetch(S, j):
            pltpu.async_copy(dstr_hbm.at[wid, j], dsts[S], sem_d[S])
            pltpu.async_copy(hs_hbm.at[src_all.at[j]], rows[S], sem_g[S])

        prefetch(0, 0)

        def pair_body(p, carry):
            for S in (0, 1):
                j = 2 * p + S
                SN = 1 - S

                @pl.when(j + 1 < NCH)
                def _():
                    @pl.when(j >= 1)
                    def _():
                        pltpu.make_async_copy(
                            rows[SN], acc_sh.at[dsts[SN]], sem_s[SN]).wait()
                    prefetch(SN, j + 1)

                pltpu.make_async_copy(dstr_hbm.at[wid, j], dsts[S],
                                      sem_d[S]).wait()
                pltpu.make_async_copy(hs_hbm.at[src_all.at[j]], rows[S],
                                      sem_g[S]).wait()
                pltpu.async_copy(rows[S], acc_sh.at[dsts[S]], sem_s[S],
                                 add=True)
            return carry

        lax.fori_loop(0, NCH // 2, pair_body, 0)
        for S in (0, 1):
            pltpu.make_async_copy(rows[S], acc_sh.at[dsts[S]], sem_s[S]).wait()
        plsc.subcore_barrier()
        pltpu.sync_copy(acc_sh.at[pl.ds(s * RT, RT)],
                        out_hbm.at[c, pl.ds(s * RT, RT)])

    return kscat


def _prep_body(x_ref, w1_ref, degp_ref, hs_ref, dinv_ref):
    x = x_ref[...]
    rowsum = jnp.sum(x, axis=1, keepdims=True)
    rinv = jnp.where(rowsum != 0.0, 1.0 / rowsum, 0.0)
    xn = x * rinv
    deg = jnp.sum(degp_ref[...], axis=0)[:, None] + 1.0
    dinv = lax.rsqrt(deg)
    h = jnp.dot(xn, w1_ref[...], preferred_element_type=jnp.float32)
    hs_ref[...] = h * dinv
    dinv_ref[...] = dinv


def _mid_body(accp_ref, hs_ref, dinv_ref, b_ref, w2_ref, hs2_ref):
    dinv = dinv_ref[...]
    out1 = (accp_ref[0] + accp_ref[1] + hs_ref[...]) * dinv + b_ref[...]
    h1 = jnp.maximum(out1, 0.0)
    h2 = jnp.dot(h1, w2_ref[...], preferred_element_type=jnp.float32)
    hs2_ref[...] = h2 * dinv


def _fin_body(accp_ref, hs_ref, dinv_ref, b_ref, out_ref):
    out_ref[...] = ((accp_ref[0] + accp_ref[1] + hs_ref[...]) * dinv_ref[...]
                    + b_ref[...])


def kernel(x, edge_index, W1, b1, W2, b2, group):
    N, D = x.shape
    H = W1.shape[1]
    E = edge_index.shape[1]

    NP = _rup(N + 1, _NS * 8)
    EW = _rup(-(-E // _NW), 4 * _CH)
    NCH = EW // _CH

    pad_e = EW * _NW - E
    pad_idx = N + jnp.arange(pad_e, dtype=jnp.int32) % (NP - N)
    src = jnp.concatenate([edge_index[0], pad_idx]).reshape(_NW, NCH, _CH)
    dst = jnp.concatenate([edge_index[1], pad_idx]).reshape(_NW, NCH, _CH)

    xpad = jnp.zeros((NP, D), jnp.float32).at[:N].set(x)
    RT = NP // _NS
    zflat = jnp.zeros((NP,), jnp.float32)
    zrowH = jnp.zeros((RT, H), jnp.float32)
    b1r = b1.reshape(1, H)
    b2r = b2.reshape(1, H)

    degp = _make_deg_kernel(NP, NCH)(dst, zflat)

    hs1, dinv = pl.pallas_call(
        _prep_body,
        out_shape=(
            jax.ShapeDtypeStruct((NP, H), jnp.float32),
            jax.ShapeDtypeStruct((NP, 1), jnp.float32),
        ),
    )(xpad, W1, degp)

    edge_scatter = _make_edge_scatter_kernel(NP, H, NCH)
    acc1 = edge_scatter(hs1, src, dst, zrowH)

    hs2 = pl.pallas_call(
        _mid_body,
        out_shape=jax.ShapeDtypeStruct((NP, H), jnp.float32),
    )(acc1, hs1, dinv, b1r, W2)

    acc2 = edge_scatter(hs2, src, dst, zrowH)

    out = pl.pallas_call(
        _fin_body,
        out_shape=jax.ShapeDtypeStruct((NP, H), jnp.float32),
    )(acc2, hs2, dinv, b2r)

    return out[:N]


# speedup vs baseline: 3.5576x; 1.0034x over previous
import functools

import jax
import jax.numpy as jnp
from jax import lax
from jax.experimental import pallas as pl
from jax.experimental.pallas import tpu as pltpu
from jax.experimental.pallas import tpu_sc as plsc

_NC = 2
_NS = 16
_NW = _NC * _NS
_CH = 128


def _rup(a, b):
    return (a + b - 1) // b * b


@functools.lru_cache(maxsize=None)
def _make_deg_kernel(NP, EWr):
    mesh = plsc.VectorSubcoreMesh(core_axis_name="c", subcore_axis_name="s")

    @functools.partial(
        pl.kernel,
        out_type=jax.ShapeDtypeStruct((_NW, NP), jnp.float32),
        mesh=mesh,
        compiler_params=pltpu.CompilerParams(needs_layout_passes=False),
        scratch_types=[
            pltpu.VMEM((EWr,), jnp.int32),
            pltpu.VMEM((NP,), jnp.float32),
        ],
    )
    def kdeg(dst_hbm, zfull_hbm, out_hbm, dst_all, deg_v):
        c = lax.axis_index("c")
        s = lax.axis_index("s")
        wid = s * _NC + c
        ones = jnp.ones((16,), jnp.float32)
        pltpu.sync_copy(zfull_hbm, deg_v)
        pltpu.sync_copy(dst_hbm.at[pl.ds(wid * EWr, EWr)], dst_all)

        def body(i, carry):
            idx = dst_all[pl.ds(i * 16, 16)]
            plsc.addupdate_scatter(deg_v, [idx], ones)
            return carry

        lax.fori_loop(0, EWr // 16, body, 0)
        pltpu.sync_copy(deg_v, out_hbm.at[wid])

    return kdeg


@functools.lru_cache(maxsize=None)
def _make_edge_scatter_kernel(NP, Hd, EWr):
    RT = NP // _NS
    NFULL = EWr // _CH
    NPIPE = NFULL // 2 * 2
    REM = EWr - NPIPE * _CH
    mesh = plsc.VectorSubcoreMesh(core_axis_name="c", subcore_axis_name="s")

    scratch = [
        pltpu.VMEM((EWr,), jnp.int32),
        pltpu.VMEM((_CH,), jnp.int32),
        pltpu.VMEM((_CH,), jnp.int32),
        pltpu.VMEM((_CH, Hd), jnp.float32),
        pltpu.VMEM((_CH, Hd), jnp.float32),
        pltpu.VMEM((max(REM, 8),), jnp.int32),
        pltpu.VMEM((max(REM, 8), Hd), jnp.float32),
        pltpu.VMEM_SHARED((NP, Hd), jnp.float32),
        pltpu.SemaphoreType.DMA,
        pltpu.SemaphoreType.DMA,
        pltpu.SemaphoreType.DMA,
        pltpu.SemaphoreType.DMA,
        pltpu.SemaphoreType.DMA,
        pltpu.SemaphoreType.DMA,
    ]

    @functools.partial(
        pl.kernel,
        out_type=jax.ShapeDtypeStruct((_NC, NP, Hd), jnp.float32),
        mesh=mesh,
        scratch_types=scratch,
    )
    def kscat(hs_hbm, src_hbm, dst_hbm, zrow_hbm, out_hbm,
              src_all, dst0, dst1, rows0, rows1, dstr_v, rowsr_v, acc_sh,
              sem_d0, sem_d1, sem_g0, sem_g1, sem_s0, sem_s1):
        c = lax.axis_index("c")
        s = lax.axis_index("s")
        wid = s * _NC + c
        base = wid * EWr
        dsts = (dst0, dst1)
        rows = (rows0, rows1)
        sem_d = (sem_d0, sem_d1)
        sem_g = (sem_g0, sem_g1)
        sem_s = (sem_s0, sem_s1)
        pltpu.sync_copy(zrow_hbm, acc_sh.at[pl.ds(s * RT, RT)])
        pltpu.sync_copy(src_hbm.at[pl.ds(base, EWr)], src_all)
        plsc.subcore_barrier()

        def prefetch(S, j):
            pltpu.async_copy(dst_hbm.at[pl.ds(base + j * _CH, _CH)],
                             dsts[S], sem_d[S])
            pltpu.async_copy(hs_hbm.at[src_all.at[pl.ds(j * _CH, _CH)]],
                             rows[S], sem_g[S])

        if NPIPE > 0:
            prefetch(0, 0)

            def pair_body(p, carry):
                for S in (0, 1):
                    j = 2 * p + S
                    SN = 1 - S

                    @pl.when(j + 1 < NPIPE)
                    def _():
                        @pl.when(j >= 1)
                        def _():
                            pltpu.make_async_copy(
                                rows[SN], acc_sh.at[dsts[SN]],
                                sem_s[SN]).wait()
                        prefetch(SN, j + 1)

                    pltpu.make_async_copy(
                        dst_hbm.at[pl.ds(base + j * _CH, _CH)], dsts[S],
                        sem_d[S]).wait()
                    pltpu.make_async_copy(
                        hs_hbm.at[src_all.at[pl.ds(j * _CH, _CH)]], rows[S],
                        sem_g[S]).wait()
                    pltpu.async_copy(rows[S], acc_sh.at[dsts[S]], sem_s[S],
                                     add=True)
                return carry

            lax.fori_loop(0, NPIPE // 2, pair_body, 0)
            for S in (0, 1):
                pltpu.make_async_copy(rows[S], acc_sh.at[dsts[S]],
                                      sem_s[S]).wait()
        if REM > 0:
            off = NPIPE * _CH
            pltpu.sync_copy(dst_hbm.at[pl.ds(base + off, REM)], dstr_v)
            pltpu.async_copy(hs_hbm.at[src_all.at[pl.ds(off, REM)]],
                             rowsr_v, sem_g0).wait()
            pltpu.sync_copy(rowsr_v, acc_sh.at[dstr_v], add=True)
        plsc.subcore_barrier()
        pltpu.sync_copy(acc_sh.at[pl.ds(s * RT, RT)],
                        out_hbm.at[c, pl.ds(s * RT, RT)])

    return kscat


def _prep_body(x_ref, w1_ref, degp_ref, hs_ref, dinv_ref):
    x = x_ref[...]
    rowsum = jnp.sum(x, axis=1, keepdims=True)
    rinv = jnp.where(rowsum != 0.0, 1.0 / rowsum, 0.0)
    xn = x * rinv
    deg = jnp.sum(degp_ref[...], axis=0)[:, None] + 1.0
    dinv = lax.rsqrt(deg)
    h = jnp.dot(xn, w1_ref[...], preferred_element_type=jnp.float32)
    hs_ref[...] = h * dinv
    dinv_ref[...] = dinv


def _mid_body(accp_ref, hs_ref, dinv_ref, b_ref, w2_ref, hs2_ref):
    dinv = dinv_ref[...]
    out1 = (accp_ref[0] + accp_ref[1] + hs_ref[...]) * dinv + b_ref[...]
    h1 = jnp.maximum(out1, 0.0)
    h2 = jnp.dot(h1, w2_ref[...], preferred_element_type=jnp.float32)
    hs2_ref[...] = h2 * dinv


def _fin_body(accp_ref, hs_ref, dinv_ref, b_ref, out_ref):
    out_ref[...] = ((accp_ref[0] + accp_ref[1] + hs_ref[...]) * dinv_ref[...]
                    + b_ref[...])


def kernel(x, edge_index, W1, b1, W2, b2, group):
    N, D = x.shape
    H = W1.shape[1]
    E = edge_index.shape[1]

    NP = _rup(N + 1, _NS * 8)
    EWr = _rup(-(-E // _NW), 16)
    EP = EWr * _NW
    srcf = edge_index[0]
    dstf = edge_index[1]
    if EP > E:
        pad_idx = N + jnp.arange(EP - E, dtype=jnp.int32) % (NP - N)
        srcf = jnp.concatenate([srcf, pad_idx])
        dstf = jnp.concatenate([dstf, pad_idx])

    xpad = jnp.zeros((NP, D), jnp.float32).at[:N].set(x)
    RT = NP // _NS
    zflat = jnp.zeros((NP,), jnp.float32)
    zrowH = jnp.zeros((RT, H), jnp.float32)
    b1r = b1.reshape(1, H)
    b2r = b2.reshape(1, H)

    degp = _make_deg_kernel(NP, EWr)(dstf, zflat)

    hs1, dinv = pl.pallas_call(
        _prep_body,
        out_shape=(
            jax.ShapeDtypeStruct((NP, H), jnp.float32),
            jax.ShapeDtypeStruct((NP, 1), jnp.float32),
        ),
    )(xpad, W1, degp)

    edge_scatter = _make_edge_scatter_kernel(NP, H, EWr)
    acc1 = edge_scatter(hs1, srcf, dstf, zrowH)

    hs2 = pl.pallas_call(
        _mid_body,
        out_shape=jax.ShapeDtypeStruct((NP, H), jnp.float32),
    )(acc1, hs1, dinv, b1r, W2)

    acc2 = edge_scatter(hs2, srcf, dstf, zrowH)

    out = pl.pallas_call(
        _fin_body,
        out_shape=jax.ShapeDtypeStruct((NP, H), jnp.float32),
    )(acc2, hs2, dinv, b2r)

    return out[:N]
